# baseline (speedup 1.0000x reference)
import jax
import jax.numpy as jnp
from jax.experimental import pallas as pl
from jax.experimental.pallas import tpu as pltpu

N = 8192
M = 8192
K = 128
BI = 1024
BJ = 1024
NI = N // BI
NJ = M // BJ


def _chamfer_tile(x_ref, y_ref, out_ref, rowmin_ref, colmin_ref):
    i = pl.program_id(0)
    j = pl.program_id(1)
    xb = x_ref[...]
    yb = y_ref[...]
    xy = jax.lax.dot_general(
        xb, yb, (((1,), (1,)), ((), ())),
        preferred_element_type=jnp.float32,
    )
    x2 = jnp.sum(xb * xb, axis=1, keepdims=True)
    y2 = jnp.sum(yb * yb, axis=1)
    d = jnp.maximum(x2 + y2[None, :] - 2.0 * xy, 0.0)
    rmin = jnp.min(d, axis=1)
    cmin = jnp.min(d, axis=0)

    @pl.when(j == 0)
    def _():
        rowmin_ref[i, :] = rmin

    @pl.when(j > 0)
    def _():
        rowmin_ref[i, :] = jnp.minimum(rowmin_ref[i, :], rmin)

    @pl.when(i == 0)
    def _():
        colmin_ref[j, :] = cmin

    @pl.when(i > 0)
    def _():
        colmin_ref[j, :] = jnp.minimum(colmin_ref[j, :], cmin)

    @pl.when((i == NI - 1) & (j == NJ - 1))
    def _():
        rsum = jnp.sum(rowmin_ref[...], keepdims=True)
        csum = jnp.sum(colmin_ref[...], keepdims=True)
        out_ref[...] = rsum + csum


def kernel(x, y):
    out = pl.pallas_call(
        _chamfer_tile,
        grid=(NI, NJ),
        in_specs=[
            pl.BlockSpec((BI, K), lambda i, j: (i, 0)),
            pl.BlockSpec((BJ, K), lambda i, j: (j, 0)),
        ],
        out_specs=pl.BlockSpec((1, 1), lambda i, j: (0, 0)),
        out_shape=jax.ShapeDtypeStruct((1, 1), jnp.float32),
        scratch_shapes=[
            pltpu.VMEM((NI, BI), jnp.float32),
            pltpu.VMEM((NJ, BJ), jnp.float32),
        ],
        compiler_params=pltpu.CompilerParams(
            dimension_semantics=("arbitrary", "arbitrary"),
        ),
    )(x, y)
    return out[0, 0]


# speedup vs baseline: 1.2995x; 1.2995x over previous
import jax
import jax.numpy as jnp
from jax.experimental import pallas as pl
from jax.experimental.pallas import tpu as pltpu

N = 8192
M = 8192
K = 128
BI = 1024
BJ = 1024
NI = N // BI
NJ = M // BJ


def _chamfer_tile(x_ref, y_ref, out_ref, rowmax_ref, colmax_ref):
    i = pl.program_id(0)
    j = pl.program_id(1)
    xb = x_ref[...]
    yb = y_ref[...]
    g = 0.5 * jnp.sum(xb * xb, axis=1, keepdims=True)
    h = 0.5 * jnp.sum(yb * yb, axis=1, keepdims=True)
    ones_i = jnp.full((BI, 1), -1.0, dtype=jnp.float32)
    ones_j = jnp.full((BJ, 1), -1.0, dtype=jnp.float32)
    xa = jnp.concatenate([xb, ones_i, g], axis=1).astype(jnp.bfloat16)
    ya = jnp.concatenate([yb, h, ones_j], axis=1).astype(jnp.bfloat16)
    p = jax.lax.dot_general(
        xa, ya, (((1,), (1,)), ((), ())),
        preferred_element_type=jnp.float32,
    )
    rmax = jnp.max(p, axis=1)
    cmax = jnp.max(p, axis=0)

    @pl.when(j == 0)
    def _():
        rowmax_ref[i, :] = rmax

    @pl.when(j > 0)
    def _():
        rowmax_ref[i, :] = jnp.maximum(rowmax_ref[i, :], rmax)

    @pl.when(i == 0)
    def _():
        colmax_ref[j, :] = cmax

    @pl.when(i > 0)
    def _():
        colmax_ref[j, :] = jnp.maximum(colmax_ref[j, :], cmax)

    @pl.when((i == NI - 1) & (j == NJ - 1))
    def _():
        d_xy = jnp.maximum(-2.0 * rowmax_ref[...], 0.0)
        d_yx = jnp.maximum(-2.0 * colmax_ref[...], 0.0)
        out_ref[...] = (jnp.sum(d_xy, keepdims=True)
                        + jnp.sum(d_yx, keepdims=True))


def kernel(x, y):
    out = pl.pallas_call(
        _chamfer_tile,
        grid=(NI, NJ),
        in_specs=[
            pl.BlockSpec((BI, K), lambda i, j: (i, 0)),
            pl.BlockSpec((BJ, K), lambda i, j: (j, 0)),
        ],
        out_specs=pl.BlockSpec((1, 1), lambda i, j: (0, 0)),
        out_shape=jax.ShapeDtypeStruct((1, 1), jnp.float32),
        scratch_shapes=[
            pltpu.VMEM((NI, BI), jnp.float32),
            pltpu.VMEM((NJ, BJ), jnp.float32),
        ],
        compiler_params=pltpu.CompilerParams(
            dimension_semantics=("arbitrary", "arbitrary"),
        ),
    )(x, y)
    return out[0, 0]


# speedup vs baseline: 1.7382x; 1.3376x over previous
import jax
import jax.numpy as jnp
from jax.experimental import pallas as pl
from jax.experimental.pallas import tpu as pltpu

N = 8192
M = 8192
K = 128
BI = 1024
BJ = 1024
NI = N // BI
NJ = M // BJ
KA = K + 2
LANE = 128
SUB = 8


def _aug_x(b):
    g = 0.5 * jnp.sum(b * b, axis=1, keepdims=True)
    neg1 = jnp.full_like(g, -1.0)
    return jnp.concatenate([b, neg1, g], axis=1).astype(jnp.bfloat16)


def _aug_y(b):
    h = 0.5 * jnp.sum(b * b, axis=1, keepdims=True)
    neg1 = jnp.full_like(h, -1.0)
    return jnp.concatenate([b, h, neg1], axis=1).astype(jnp.bfloat16)


def _chamfer_tile(x_ref, y_ref, out_ref, xa_s, ya_s, rowacc, colacc):
    i = pl.program_id(0)
    j = pl.program_id(1)

    @pl.when(j == 0)
    def _():
        xa_s[...] = _aug_x(x_ref[...])

    @pl.when(i == 0)
    def _():
        ya_s[j] = _aug_y(y_ref[...])

    p = jax.lax.dot_general(
        xa_s[...], ya_s[j], (((1,), (1,)), ((), ())),
        preferred_element_type=jnp.float32,
    )

    pr = p[:, 0:LANE]
    for c in range(1, BJ // LANE):
        pr = jnp.maximum(pr, p[:, c * LANE:(c + 1) * LANE])
    pc = p[0:SUB, :]
    for r in range(1, BI // SUB):
        pc = jnp.maximum(pc, p[r * SUB:(r + 1) * SUB, :])

    @pl.when(j == 0)
    def _():
        rowacc[i] = pr

    @pl.when(j > 0)
    def _():
        rowacc[i] = jnp.maximum(rowacc[i], pr)

    @pl.when(i == 0)
    def _():
        colacc[j] = pc

    @pl.when(i > 0)
    def _():
        colacc[j] = jnp.maximum(colacc[j], pc)

    @pl.when((i == NI - 1) & (j == NJ - 1))
    def _():
        rm = jnp.max(rowacc[...], axis=2)
        d_xy = jnp.maximum(-2.0 * rm, 0.0)
        cm = jnp.max(colacc[...], axis=1)
        d_yx = jnp.maximum(-2.0 * cm, 0.0)
        out_ref[...] = (jnp.sum(d_xy, keepdims=True)
                        + jnp.sum(d_yx, keepdims=True))


def kernel(x, y):
    out = pl.pallas_call(
        _chamfer_tile,
        grid=(NI, NJ),
        in_specs=[
            pl.BlockSpec((BI, K), lambda i, j: (i, 0)),
            pl.BlockSpec((BJ, K), lambda i, j: (j, 0)),
        ],
        out_specs=pl.BlockSpec((1, 1), lambda i, j: (0, 0)),
        out_shape=jax.ShapeDtypeStruct((1, 1), jnp.float32),
        scratch_shapes=[
            pltpu.VMEM((BI, KA), jnp.bfloat16),
            pltpu.VMEM((NJ, BJ, KA), jnp.bfloat16),
            pltpu.VMEM((NI, BI, LANE), jnp.float32),
            pltpu.VMEM((NJ, SUB, BJ), jnp.float32),
        ],
        compiler_params=pltpu.CompilerParams(
            dimension_semantics=("arbitrary", "arbitrary"),
        ),
    )(x, y)
    return out[0, 0]


# speedup vs baseline: 1.7615x; 1.0134x over previous
import jax
import jax.numpy as jnp
from jax.experimental import pallas as pl
from jax.experimental.pallas import tpu as pltpu

N = 8192
M = 8192
K = 128
BI = 1024
BJ = 1024
NI = N // BI
NJ = M // BJ
KA = K + 2
LANE = 128
SUB = 8


def _aug_x(b):
    g = 0.5 * jnp.sum(b * b, axis=1, keepdims=True)
    neg1 = jnp.full_like(g, -1.0)
    return jnp.concatenate([b, neg1, g], axis=1).astype(jnp.bfloat16)


def _aug_y(b):
    h = 0.5 * jnp.sum(b * b, axis=1, keepdims=True)
    neg1 = jnp.full_like(h, -1.0)
    return jnp.concatenate([b, h, neg1], axis=1).astype(jnp.bfloat16)


def _chamfer_tile(x_ref, y_ref, out_ref, xa_s, ya_s, rowacc, colacc):
    i = pl.program_id(0)
    j = pl.program_id(1)

    @pl.when(j == 0)
    def _():
        xa_s[...] = _aug_x(x_ref[...])

    @pl.when(i == 0)
    def _():
        ya_s[j] = _aug_y(y_ref[...])

    p = jax.lax.dot_general(
        xa_s[...], ya_s[j], (((1,), (1,)), ((), ())),
        preferred_element_type=jnp.float32,
    ).astype(jnp.bfloat16)

    pr = p[:, 0:LANE]
    for c in range(1, BJ // LANE):
        pr = jnp.maximum(pr, p[:, c * LANE:(c + 1) * LANE])
    pc = p[0:SUB, :]
    for r in range(1, BI // SUB):
        pc = jnp.maximum(pc, p[r * SUB:(r + 1) * SUB, :])

    @pl.when(j == 0)
    def _():
        rowacc[i] = pr

    @pl.when(j > 0)
    def _():
        rowacc[i] = jnp.maximum(rowacc[i], pr)

    @pl.when(i == 0)
    def _():
        colacc[j] = pc

    @pl.when(i > 0)
    def _():
        colacc[j] = jnp.maximum(colacc[j], pc)

    @pl.when((i == NI - 1) & (j == NJ - 1))
    def _():
        rm = jnp.max(rowacc[...], axis=2).astype(jnp.float32)
        d_xy = jnp.maximum(-2.0 * rm, 0.0)
        cm = jnp.max(colacc[...], axis=1).astype(jnp.float32)
        d_yx = jnp.maximum(-2.0 * cm, 0.0)
        out_ref[...] = (jnp.sum(d_xy, keepdims=True)
                        + jnp.sum(d_yx, keepdims=True))


def kernel(x, y):
    out = pl.pallas_call(
        _chamfer_tile,
        grid=(NI, NJ),
        in_specs=[
            pl.BlockSpec((BI, K), lambda i, j: (i, 0)),
            pl.BlockSpec((BJ, K), lambda i, j: (j, 0)),
        ],
        out_specs=pl.BlockSpec((1, 1), lambda i, j: (0, 0)),
        out_shape=jax.ShapeDtypeStruct((1, 1), jnp.float32),
        scratch_shapes=[
            pltpu.VMEM((BI, KA), jnp.bfloat16),
            pltpu.VMEM((NJ, BJ, KA), jnp.bfloat16),
            pltpu.VMEM((NI, BI, LANE), jnp.bfloat16),
            pltpu.VMEM((NJ, SUB, BJ), jnp.bfloat16),
        ],
        compiler_params=pltpu.CompilerParams(
            dimension_semantics=("arbitrary", "arbitrary"),
        ),
    )(x, y)
    return out[0, 0]
